# baseline (speedup 1.0000x reference)
import functools

import jax
import jax.numpy as jnp
from jax import lax
from jax.experimental import pallas as pl
from jax.experimental.pallas import tpu as pltpu
from jax.experimental.pallas import tpu_sc as plsc

BATCH = 16384
EMB_D = 16
NC = 2
NS = 16
NW = NC * NS
B_PER_W = BATCH // NW
CHUNK = 128
NCHUNK = B_PER_W // CHUNK
ROWS_MAJ = BATCH // CHUNK


def _gather_body(uidx_hbm, iidx_hbm, uemb_hbm, iemb_hbm, uout_hbm, iout_hbm,
                 uidx_v, iidx_v, urows_v, irows_v, usem, isem):
    wid = lax.axis_index("s") * NC + lax.axis_index("c")
    base = wid * NCHUNK
    pltpu.sync_copy(uidx_hbm.at[pl.ds(base, NCHUNK)], uidx_v)
    pltpu.sync_copy(iidx_hbm.at[pl.ds(base, NCHUNK)], iidx_v)
    ucopies = []
    icopies = []
    for j in range(NCHUNK):
        ucopies.append(pltpu.async_copy(uemb_hbm.at[uidx_v.at[j]], urows_v.at[j], usem))
        icopies.append(pltpu.async_copy(iemb_hbm.at[iidx_v.at[j]], irows_v.at[j], isem))
    for c in ucopies:
        c.wait()
    pltpu.sync_copy(urows_v, uout_hbm.at[pl.ds(base, NCHUNK)])
    for c in icopies:
        c.wait()
    pltpu.sync_copy(irows_v, iout_hbm.at[pl.ds(base, NCHUNK)])


@jax.jit
def _sc_gather(uidx, iidx, uemb, iemb):
    mesh = plsc.VectorSubcoreMesh(core_axis_name="c", subcore_axis_name="s")
    out_t = (jax.ShapeDtypeStruct((ROWS_MAJ, CHUNK, EMB_D), jnp.float32),
             jax.ShapeDtypeStruct((ROWS_MAJ, CHUNK, EMB_D), jnp.float32))
    fn = functools.partial(
        pl.kernel, mesh=mesh, out_type=out_t,
        compiler_params=pltpu.CompilerParams(use_tc_tiling_on_sc=False),
        scratch_types=[
            pltpu.VMEM((NCHUNK, CHUNK), jnp.int32),
            pltpu.VMEM((NCHUNK, CHUNK), jnp.int32),
            pltpu.VMEM((NCHUNK, CHUNK, EMB_D), jnp.float32),
            pltpu.VMEM((NCHUNK, CHUNK, EMB_D), jnp.float32),
            pltpu.SemaphoreType.DMA,
            pltpu.SemaphoreType.DMA,
        ],
    )(_gather_body)
    return fn(uidx, iidx, uemb, iemb)


def _mlp_body(u_ref, i_ref, w1u_ref, w1i_ref, b1_ref, w2_ref, b2_ref,
              w3_ref, b3_ref, o_ref):
    h = (jnp.dot(u_ref[...], w1u_ref[...], preferred_element_type=jnp.float32)
         + jnp.dot(i_ref[...], w1i_ref[...], preferred_element_type=jnp.float32)
         + b1_ref[...])
    h = jnp.maximum(h, 0.0)
    h = jnp.maximum(
        jnp.dot(h, w2_ref[...], preferred_element_type=jnp.float32) + b2_ref[...],
        0.0)
    o_ref[...] = (jnp.dot(h, w3_ref[...], preferred_element_type=jnp.float32)
                  + b3_ref[...])


@jax.jit
def _tc_mlp(u_lat, i_lat, w1u, w1i, b1, w2t, b2, w3t, b3):
    grid = 8
    blk = BATCH // grid
    full = lambda shape: pl.BlockSpec(shape, lambda g: (0,) * len(shape))
    return pl.pallas_call(
        _mlp_body,
        grid=(grid,),
        in_specs=[
            pl.BlockSpec((blk, EMB_D), lambda g: (g, 0)),
            pl.BlockSpec((blk, EMB_D), lambda g: (g, 0)),
            full((EMB_D, 16)), full((EMB_D, 16)), full((1, 16)),
            full((16, 8)), full((1, 8)),
            full((8, 1)), full((1, 1)),
        ],
        out_specs=pl.BlockSpec((blk, 1), lambda g: (g, 0)),
        out_shape=jax.ShapeDtypeStruct((BATCH, 1), jnp.float32),
    )(u_lat, i_lat, w1u, w1i, b1, w2t, b2, w3t, b3)


def kernel(user_input, item_input, user_emb, item_emb, W1, b1, W2, b2, W3, b3):
    uidx = user_input.astype(jnp.int32).reshape(ROWS_MAJ, CHUNK)
    iidx = item_input.astype(jnp.int32).reshape(ROWS_MAJ, CHUNK)
    u3, i3 = _sc_gather(uidx, iidx, user_emb, item_emb)
    u_lat = u3.reshape(BATCH, EMB_D)
    i_lat = i3.reshape(BATCH, EMB_D)
    w1u = W1[:, :EMB_D].T
    w1i = W1[:, EMB_D:].T
    w2t = W2.T
    w3t = W3.T
    return _tc_mlp(u_lat, i_lat, w1u, w1i, b1.reshape(1, 16), w2t,
                   b2.reshape(1, 8), w3t, b3.reshape(1, 1))


# speedup vs baseline: 5.3145x; 5.3145x over previous
import functools

import jax
import jax.numpy as jnp
from jax import lax
from jax.experimental import pallas as pl
from jax.experimental.pallas import tpu as pltpu
from jax.experimental.pallas import tpu_sc as plsc

BATCH = 16384
EMB_D = 16
NROWS = 1000000
NC = 2
NS = 16
NW = NC * NS
B_PER_W = BATCH // NW
KF = 16
NGRP = B_PER_W // KF
LAST_BLK = NROWS // 128 - 1
EDGE = (NROWS // 128) * 128
EDGE_W = NROWS - EDGE


def _gather_body(idx_hbm, uembT_hbm, iembT_hbm, uedge_hbm, iedge_hbm,
                 uoutT_hbm, ioutT_hbm,
                 idx_v, ublk_v, iblk_v, urows_v, irows_v, edge_v,
                 usem, isem, esem):
    wid = lax.axis_index("s") * NC + lax.axis_index("c")
    base = wid * B_PER_W
    pltpu.sync_copy(idx_hbm.at[wid], idx_v)
    pltpu.async_copy(uedge_hbm, edge_v.at[0], esem).wait()
    pltpu.async_copy(iedge_hbm, edge_v.at[1], esem).wait()

    iota16 = lax.iota(jnp.int32, 16)

    def splat(x):
        return jnp.full((16,), x, jnp.int32)

    def ldv(toff, g):
        return idx_v[toff + (g >> 3), pl.ds(16 * (g & 7), 16)]

    def fire(toff, table, blk_v, sem, g):
        v = ldv(toff, g)
        for k in range(KF):
            r = v[k]
            cb = jnp.minimum(r >> 7, LAST_BLK)
            pltpu.async_copy(table.at[:, pl.ds(cb * 128, 128)],
                             blk_v.at[k], sem)

    def drain_extract(toff, table, blk_v, rows_v, sem, g):
        v = ldv(toff, g)
        for k in range(KF):
            pltpu.make_async_copy(table.at[:, pl.ds(0, 128)],
                                  blk_v.at[k], sem).wait()
        for k in range(KF):
            i = g * KF + k
            x = plsc.load_gather(blk_v, [splat(k), iota16,
                                         splat(v[k] & 127)])
            plsc.store_scatter(rows_v, [iota16, splat(i)], x)

    fire(0, uembT_hbm, ublk_v, usem, 0)

    def grp(g, _):
        fire(4, iembT_hbm, iblk_v, isem, g)
        drain_extract(0, uembT_hbm, ublk_v, urows_v, usem, g)
        @pl.when(g < NGRP - 1)
        def _():
            fire(0, uembT_hbm, ublk_v, usem, g + 1)
        drain_extract(4, iembT_hbm, iblk_v, irows_v, isem, g)
        return 0

    lax.fori_loop(0, NGRP, grp, 0)

    def tail(g, _):
        vu = ldv(0, g)
        vi = ldv(4, g)
        for k in range(KF):
            i = g * KF + k
            ru = vu[k]
            @pl.when(ru >= EDGE)
            def _():
                x = plsc.load_gather(edge_v, [splat(0), iota16,
                                              splat(ru - EDGE)])
                plsc.store_scatter(urows_v, [iota16, splat(i)], x)
            ri = vi[k]
            @pl.when(ri >= EDGE)
            def _():
                x = plsc.load_gather(edge_v, [splat(1), iota16,
                                              splat(ri - EDGE)])
                plsc.store_scatter(irows_v, [iota16, splat(i)], x)
        return 0

    lax.fori_loop(0, NGRP, tail, 0)

    pltpu.sync_copy(urows_v, uoutT_hbm.at[:, pl.ds(base, B_PER_W)])
    pltpu.sync_copy(irows_v, ioutT_hbm.at[:, pl.ds(base, B_PER_W)])


@jax.jit
def _sc_gather(idx, uembT, iembT, uedge, iedge):
    mesh = plsc.VectorSubcoreMesh(core_axis_name="c", subcore_axis_name="s")
    out_t = (jax.ShapeDtypeStruct((EMB_D, BATCH), jnp.float32),
             jax.ShapeDtypeStruct((EMB_D, BATCH), jnp.float32))
    fn = functools.partial(
        pl.kernel, mesh=mesh, out_type=out_t,
        compiler_params=pltpu.CompilerParams(needs_layout_passes=False),
        scratch_types=[
            pltpu.VMEM((8, 128), jnp.int32),
            pltpu.VMEM((KF, EMB_D, 128), jnp.float32),
            pltpu.VMEM((KF, EMB_D, 128), jnp.float32),
            pltpu.VMEM((EMB_D, B_PER_W), jnp.float32),
            pltpu.VMEM((EMB_D, B_PER_W), jnp.float32),
            pltpu.VMEM((2, EMB_D, EDGE_W), jnp.float32),
            pltpu.SemaphoreType.DMA,
            pltpu.SemaphoreType.DMA,
            pltpu.SemaphoreType.DMA,
        ],
    )(_gather_body)
    return fn(idx, uembT, iembT, uedge, iedge)


def _mlp_body(u_ref, i_ref, w1u_ref, w1i_ref, b1_ref, w2_ref, b2_ref,
              w3_ref, b3_ref, o_ref):
    h = (jnp.dot(w1u_ref[...], u_ref[...], preferred_element_type=jnp.float32)
         + jnp.dot(w1i_ref[...], i_ref[...], preferred_element_type=jnp.float32)
         + b1_ref[...])
    h = jnp.maximum(h, 0.0)
    h = jnp.maximum(
        jnp.dot(w2_ref[...], h, preferred_element_type=jnp.float32) + b2_ref[...],
        0.0)
    o_ref[...] = (jnp.dot(w3_ref[...], h, preferred_element_type=jnp.float32)
                  + b3_ref[...])


@jax.jit
def _tc_mlp(uT, iT, w1u, w1i, b1, w2, b2, w3, b3):
    grid = 8
    blk = BATCH // grid
    full = lambda shape: pl.BlockSpec(shape, lambda g: (0,) * len(shape))
    return pl.pallas_call(
        _mlp_body,
        grid=(grid,),
        in_specs=[
            pl.BlockSpec((EMB_D, blk), lambda g: (0, g)),
            pl.BlockSpec((EMB_D, blk), lambda g: (0, g)),
            full((16, EMB_D)), full((16, EMB_D)), full((16, 1)),
            full((8, 16)), full((8, 1)),
            full((1, 8)), full((1, 1)),
        ],
        out_specs=pl.BlockSpec((1, blk), lambda g: (0, g)),
        out_shape=jax.ShapeDtypeStruct((1, BATCH), jnp.float32),
    )(uT, iT, w1u, w1i, b1, w2, b2, w3, b3)


def kernel(user_input, item_input, user_emb, item_emb, W1, b1, W2, b2, W3, b3):
    uidx = user_input.astype(jnp.int32).reshape(NW, 4, 128)
    iidx = item_input.astype(jnp.int32).reshape(NW, 4, 128)
    idx = jnp.concatenate([uidx, iidx], axis=1)
    uembT = user_emb.T
    iembT = item_emb.T
    uedge = uembT[:, EDGE:]
    iedge = iembT[:, EDGE:]
    uT, iT = _sc_gather(idx, uembT, iembT, uedge, iedge)
    w1u = W1[:, :EMB_D]
    w1i = W1[:, EMB_D:]
    outT = _tc_mlp(uT, iT, w1u, w1i, b1.reshape(16, 1), W2,
                   b2.reshape(8, 1), W3, b3.reshape(1, 1))
    return outT.reshape(BATCH, 1)


# speedup vs baseline: 5.5500x; 1.0443x over previous
import functools

import jax
import jax.numpy as jnp
from jax import lax
from jax.experimental import pallas as pl
from jax.experimental.pallas import tpu as pltpu
from jax.experimental.pallas import tpu_sc as plsc

BATCH = 16384
EMB_D = 16
NROWS = 1000000
NC = 2
NS = 16
NW = NC * NS
B_PER_W = BATCH // NW
KF = 16
NGRP = B_PER_W // KF
LAST_BLK = NROWS // 128 - 1
EDGE = (NROWS // 128) * 128
EDGE_W = NROWS - EDGE


def _gather_body(idx_hbm, uembT_hbm, iembT_hbm, uedge_hbm, iedge_hbm,
                 uoutT_hbm, ioutT_hbm,
                 idx_v, ublk_v, iblk_v, urows_v, irows_v, edge_v,
                 usem, isem, esem):
    wid = lax.axis_index("s") * NC + lax.axis_index("c")
    base = wid * B_PER_W
    pltpu.sync_copy(idx_hbm.at[wid], idx_v)
    pltpu.async_copy(uedge_hbm, edge_v.at[0], esem).wait()
    pltpu.async_copy(iedge_hbm, edge_v.at[1], esem).wait()

    iota16 = lax.iota(jnp.int32, 16)

    def splat(x):
        return jnp.full((16,), x, jnp.int32)

    def ldv(toff, g):
        return idx_v[toff + (g >> 3), pl.ds(16 * (g & 7), 16)]

    def fire(toff, table, blk_v, sem, g):
        v = ldv(toff, g)
        for k in range(KF):
            r = v[k]
            cb = jnp.minimum(r >> 7, LAST_BLK)
            pltpu.async_copy(table.at[:, pl.ds(cb * 128, 128)],
                             blk_v.at[k], sem)

    def drain_extract(toff, table, blk_v, rows_v, sem, g):
        v = ldv(toff, g)
        for k in range(KF):
            pltpu.make_async_copy(table.at[:, pl.ds(0, 128)],
                                  blk_v.at[k], sem).wait()
        for k in range(KF):
            i = g * KF + k
            x = plsc.load_gather(blk_v, [splat(k), iota16,
                                         splat(v[k] & 127)])
            plsc.store_scatter(rows_v, [iota16, splat(i)], x)

    fire(0, uembT_hbm, ublk_v, usem, 0)

    def grp(g, _):
        fire(4, iembT_hbm, iblk_v, isem, g)
        drain_extract(0, uembT_hbm, ublk_v, urows_v, usem, g)
        @pl.when(g < NGRP - 1)
        def _():
            fire(0, uembT_hbm, ublk_v, usem, g + 1)
        drain_extract(4, iembT_hbm, iblk_v, irows_v, isem, g)
        return 0

    lax.fori_loop(0, NGRP, grp, 0)

    def tail(g, _):
        vu = ldv(0, g)
        vi = ldv(4, g)
        @pl.when(jnp.any(vu >= EDGE))
        def _():
            for k in range(KF):
                i = g * KF + k
                ru = vu[k]
                @pl.when(ru >= EDGE)
                def _():
                    x = plsc.load_gather(edge_v, [splat(0), iota16,
                                                  splat(ru - EDGE)])
                    plsc.store_scatter(urows_v, [iota16, splat(i)], x)
        @pl.when(jnp.any(vi >= EDGE))
        def _():
            for k in range(KF):
                i = g * KF + k
                ri = vi[k]
                @pl.when(ri >= EDGE)
                def _():
                    x = plsc.load_gather(edge_v, [splat(1), iota16,
                                                  splat(ri - EDGE)])
                    plsc.store_scatter(irows_v, [iota16, splat(i)], x)
        return 0

    lax.fori_loop(0, NGRP, tail, 0)

    pltpu.sync_copy(urows_v, uoutT_hbm.at[:, pl.ds(base, B_PER_W)])
    pltpu.sync_copy(irows_v, ioutT_hbm.at[:, pl.ds(base, B_PER_W)])


@jax.jit
def _sc_gather(idx, uembT, iembT, uedge, iedge):
    mesh = plsc.VectorSubcoreMesh(core_axis_name="c", subcore_axis_name="s")
    out_t = (jax.ShapeDtypeStruct((EMB_D, BATCH), jnp.float32),
             jax.ShapeDtypeStruct((EMB_D, BATCH), jnp.float32))
    fn = functools.partial(
        pl.kernel, mesh=mesh, out_type=out_t,
        compiler_params=pltpu.CompilerParams(needs_layout_passes=False),
        scratch_types=[
            pltpu.VMEM((8, 128), jnp.int32),
            pltpu.VMEM((KF, EMB_D, 128), jnp.float32),
            pltpu.VMEM((KF, EMB_D, 128), jnp.float32),
            pltpu.VMEM((EMB_D, B_PER_W), jnp.float32),
            pltpu.VMEM((EMB_D, B_PER_W), jnp.float32),
            pltpu.VMEM((2, EMB_D, EDGE_W), jnp.float32),
            pltpu.SemaphoreType.DMA,
            pltpu.SemaphoreType.DMA,
            pltpu.SemaphoreType.DMA,
        ],
    )(_gather_body)
    return fn(idx, uembT, iembT, uedge, iedge)


def _mlp_body(u_ref, i_ref, w1u_ref, w1i_ref, b1_ref, w2_ref, b2_ref,
              w3_ref, b3_ref, o_ref):
    h = (jnp.dot(w1u_ref[...], u_ref[...], preferred_element_type=jnp.float32)
         + jnp.dot(w1i_ref[...], i_ref[...], preferred_element_type=jnp.float32)
         + b1_ref[...])
    h = jnp.maximum(h, 0.0)
    h = jnp.maximum(
        jnp.dot(w2_ref[...], h, preferred_element_type=jnp.float32) + b2_ref[...],
        0.0)
    o_ref[...] = (jnp.dot(w3_ref[...], h, preferred_element_type=jnp.float32)
                  + b3_ref[...])


@jax.jit
def _tc_mlp(uT, iT, w1u, w1i, b1, w2, b2, w3, b3):
    grid = 8
    blk = BATCH // grid
    full = lambda shape: pl.BlockSpec(shape, lambda g: (0,) * len(shape))
    return pl.pallas_call(
        _mlp_body,
        grid=(grid,),
        in_specs=[
            pl.BlockSpec((EMB_D, blk), lambda g: (0, g)),
            pl.BlockSpec((EMB_D, blk), lambda g: (0, g)),
            full((16, EMB_D)), full((16, EMB_D)), full((16, 1)),
            full((8, 16)), full((8, 1)),
            full((1, 8)), full((1, 1)),
        ],
        out_specs=pl.BlockSpec((1, blk), lambda g: (0, g)),
        out_shape=jax.ShapeDtypeStruct((1, BATCH), jnp.float32),
    )(uT, iT, w1u, w1i, b1, w2, b2, w3, b3)


def kernel(user_input, item_input, user_emb, item_emb, W1, b1, W2, b2, W3, b3):
    uidx = user_input.astype(jnp.int32).reshape(NW, 4, 128)
    iidx = item_input.astype(jnp.int32).reshape(NW, 4, 128)
    idx = jnp.concatenate([uidx, iidx], axis=1)
    uembT = user_emb.T
    iembT = item_emb.T
    uedge = uembT[:, EDGE:]
    iedge = iembT[:, EDGE:]
    uT, iT = _sc_gather(idx, uembT, iembT, uedge, iedge)
    w1u = W1[:, :EMB_D]
    w1i = W1[:, EMB_D:]
    outT = _tc_mlp(uT, iT, w1u, w1i, b1.reshape(16, 1), W2,
                   b2.reshape(8, 1), W3, b3.reshape(1, 1))
    return outT.reshape(BATCH, 1)


# speedup vs baseline: 5.5741x; 1.0043x over previous
import functools

import jax
import jax.numpy as jnp
from jax import lax
from jax.experimental import pallas as pl
from jax.experimental.pallas import tpu as pltpu
from jax.experimental.pallas import tpu_sc as plsc

BATCH = 16384
EMB_D = 16
NROWS = 1000000
NC = 2
NS = 16
NW = NC * NS
B_PER_W = BATCH // NW
KF = 16
NGRP = B_PER_W // KF
LAST_BLK = NROWS // 128 - 1
EDGE = (NROWS // 128) * 128
EDGE_W = NROWS - EDGE


def _gather_body(uidx_hbm, iidx_hbm, uembT_hbm, iembT_hbm,
                 uoutT_hbm, ioutT_hbm,
                 idx_v, ublk_v, iblk_v, urows_v, irows_v, edge_v,
                 usem, isem, esem):
    wid = lax.axis_index("s") * NC + lax.axis_index("c")
    base = wid * B_PER_W
    pltpu.sync_copy(uidx_hbm.at[pl.ds(base, B_PER_W)], idx_v.at[0])
    pltpu.sync_copy(iidx_hbm.at[pl.ds(base, B_PER_W)], idx_v.at[1])
    pltpu.async_copy(uembT_hbm.at[:, pl.ds(EDGE, EDGE_W)], edge_v.at[0],
                     esem).wait()
    pltpu.async_copy(iembT_hbm.at[:, pl.ds(EDGE, EDGE_W)], edge_v.at[1],
                     esem).wait()

    iota16 = lax.iota(jnp.int32, 16)

    def splat(x):
        return jnp.full((16,), x, jnp.int32)

    def ldv(toff, g):
        return idx_v[toff, pl.ds(16 * g, 16)]

    def fire(toff, table, blk_v, sem, g):
        v = ldv(toff, g)
        for k in range(KF):
            r = v[k]
            cb = jnp.minimum(r >> 7, LAST_BLK)
            pltpu.async_copy(table.at[:, pl.ds(cb * 128, 128)],
                             blk_v.at[k], sem)

    def drain_extract(toff, table, blk_v, rows_v, sem, g):
        v = ldv(toff, g)
        for k in range(KF):
            pltpu.make_async_copy(table.at[:, pl.ds(0, 128)],
                                  blk_v.at[k], sem).wait()
        for k in range(KF):
            i = g * KF + k
            x = plsc.load_gather(blk_v, [splat(k), iota16,
                                         splat(v[k] & 127)])
            plsc.store_scatter(rows_v, [iota16, splat(i)], x)

    fire(0, uembT_hbm, ublk_v, usem, 0)

    def grp(g, _):
        fire(1, iembT_hbm, iblk_v, isem, g)
        drain_extract(0, uembT_hbm, ublk_v, urows_v, usem, g)
        @pl.when(g < NGRP - 1)
        def _():
            fire(0, uembT_hbm, ublk_v, usem, g + 1)
        drain_extract(1, iembT_hbm, iblk_v, irows_v, isem, g)
        return 0

    lax.fori_loop(0, NGRP, grp, 0)

    def tail(g, _):
        vu = ldv(0, g)
        vi = ldv(1, g)
        @pl.when(jnp.any(vu >= EDGE))
        def _():
            for k in range(KF):
                i = g * KF + k
                ru = vu[k]
                @pl.when(ru >= EDGE)
                def _():
                    x = plsc.load_gather(edge_v, [splat(0), iota16,
                                                  splat(ru - EDGE)])
                    plsc.store_scatter(urows_v, [iota16, splat(i)], x)
        @pl.when(jnp.any(vi >= EDGE))
        def _():
            for k in range(KF):
                i = g * KF + k
                ri = vi[k]
                @pl.when(ri >= EDGE)
                def _():
                    x = plsc.load_gather(edge_v, [splat(1), iota16,
                                                  splat(ri - EDGE)])
                    plsc.store_scatter(irows_v, [iota16, splat(i)], x)
        return 0

    lax.fori_loop(0, NGRP, tail, 0)

    pltpu.sync_copy(urows_v, uoutT_hbm.at[:, pl.ds(base, B_PER_W)])
    pltpu.sync_copy(irows_v, ioutT_hbm.at[:, pl.ds(base, B_PER_W)])


@jax.jit
def _sc_gather(uidx, iidx, uembT, iembT):
    mesh = plsc.VectorSubcoreMesh(core_axis_name="c", subcore_axis_name="s")
    out_t = (jax.ShapeDtypeStruct((EMB_D, BATCH), jnp.float32),
             jax.ShapeDtypeStruct((EMB_D, BATCH), jnp.float32))
    fn = functools.partial(
        pl.kernel, mesh=mesh, out_type=out_t,
        compiler_params=pltpu.CompilerParams(needs_layout_passes=False),
        scratch_types=[
            pltpu.VMEM((2, B_PER_W), jnp.int32),
            pltpu.VMEM((KF, EMB_D, 128), jnp.float32),
            pltpu.VMEM((KF, EMB_D, 128), jnp.float32),
            pltpu.VMEM((EMB_D, B_PER_W), jnp.float32),
            pltpu.VMEM((EMB_D, B_PER_W), jnp.float32),
            pltpu.VMEM((2, EMB_D, EDGE_W), jnp.float32),
            pltpu.SemaphoreType.DMA,
            pltpu.SemaphoreType.DMA,
            pltpu.SemaphoreType.DMA,
        ],
    )(_gather_body)
    return fn(uidx, iidx, uembT, iembT)


def _mlp_body(u_ref, i_ref, w1u_ref, w1i_ref, b1_ref, w2_ref, b2_ref,
              w3_ref, b3_ref, o_ref):
    h = (jnp.dot(w1u_ref[...], u_ref[...], preferred_element_type=jnp.float32)
         + jnp.dot(w1i_ref[...], i_ref[...], preferred_element_type=jnp.float32)
         + b1_ref[...])
    h = jnp.maximum(h, 0.0)
    h = jnp.maximum(
        jnp.dot(w2_ref[...], h, preferred_element_type=jnp.float32) + b2_ref[...],
        0.0)
    o_ref[...] = (jnp.dot(w3_ref[...], h, preferred_element_type=jnp.float32)
                  + b3_ref[...])


@jax.jit
def _tc_mlp(uT, iT, w1u, w1i, b1, w2, b2, w3, b3):
    grid = 8
    blk = BATCH // grid
    full = lambda shape: pl.BlockSpec(shape, lambda g: (0,) * len(shape))
    return pl.pallas_call(
        _mlp_body,
        grid=(grid,),
        in_specs=[
            pl.BlockSpec((EMB_D, blk), lambda g: (0, g)),
            pl.BlockSpec((EMB_D, blk), lambda g: (0, g)),
            full((16, EMB_D)), full((16, EMB_D)), full((16, 1)),
            full((8, 16)), full((8, 1)),
            full((1, 8)), full((1, 1)),
        ],
        out_specs=pl.BlockSpec((1, blk), lambda g: (0, g)),
        out_shape=jax.ShapeDtypeStruct((1, BATCH), jnp.float32),
    )(uT, iT, w1u, w1i, b1, w2, b2, w3, b3)


def kernel(user_input, item_input, user_emb, item_emb, W1, b1, W2, b2, W3, b3):
    uidx = user_input.astype(jnp.int32)
    iidx = item_input.astype(jnp.int32)
    uT, iT = _sc_gather(uidx, iidx, user_emb.T, item_emb.T)
    w1u = W1[:, :EMB_D]
    w1i = W1[:, EMB_D:]
    outT = _tc_mlp(uT, iT, w1u, w1i, b1.reshape(16, 1), W2,
                   b2.reshape(8, 1), W3, b3.reshape(1, 1))
    return outT.reshape(BATCH, 1)


# speedup vs baseline: 5.7020x; 1.0229x over previous
import functools

import jax
import jax.numpy as jnp
from jax import lax
from jax.experimental import pallas as pl
from jax.experimental.pallas import tpu as pltpu
from jax.experimental.pallas import tpu_sc as plsc

BATCH = 16384
EMB_D = 16
NROWS = 1000000
NC = 2
NS = 16
NW = NC * NS
B_PER_W = BATCH // NW
KF = 16
NGRP = B_PER_W // KF
LAST_BLK = NROWS // 128 - 1
EDGE = (NROWS // 128) * 128
EDGE_W = NROWS - EDGE


def _gather_body(uidx_hbm, iidx_hbm, uembT_hbm, iembT_hbm,
                 uoutT_hbm, ioutT_hbm,
                 idx_v, ublk_v, iblk_v, urows_v, irows_v, edge_v,
                 usem, isem, esem):
    wid = lax.axis_index("s") * NC + lax.axis_index("c")
    base = wid * B_PER_W
    pltpu.sync_copy(uidx_hbm.at[pl.ds(base, B_PER_W)], idx_v.at[0])
    pltpu.sync_copy(iidx_hbm.at[pl.ds(base, B_PER_W)], idx_v.at[1])
    pltpu.async_copy(uembT_hbm.at[:, pl.ds(EDGE, EDGE_W)], edge_v.at[0],
                     esem).wait()
    pltpu.async_copy(iembT_hbm.at[:, pl.ds(EDGE, EDGE_W)], edge_v.at[1],
                     esem).wait()

    iota16 = lax.iota(jnp.int32, 16)

    def splat(x):
        return jnp.full((16,), x, jnp.int32)

    def ldv(toff, g):
        return idx_v[toff, pl.ds(16 * g, 16)]

    def fire(toff, table, blk_v, sem, g):
        v = ldv(toff, g)
        for k in range(KF):
            r = v[k]
            cb = jnp.minimum(r >> 7, LAST_BLK)
            pltpu.async_copy(table.at[:, pl.ds(cb * 128, 128)],
                             blk_v.at[k], sem)

    def drain_extract(toff, table, blk_v, rows_v, sem, g):
        v = ldv(toff, g)
        for k in range(KF):
            pltpu.make_async_copy(table.at[:, pl.ds(0, 128)],
                                  blk_v.at[k], sem).wait()
        for k in range(KF):
            i = g * KF + k
            x = plsc.load_gather(blk_v, [splat(k), iota16,
                                         splat(v[k] & 127)])
            plsc.store_scatter(rows_v, [iota16, splat(i)], x)

    fire(0, uembT_hbm, ublk_v, usem, 0)

    def grp(g, _):
        fire(1, iembT_hbm, iblk_v, isem, g)
        drain_extract(0, uembT_hbm, ublk_v, urows_v, usem, g)
        @pl.when(g < NGRP - 1)
        def _():
            fire(0, uembT_hbm, ublk_v, usem, g + 1)
        drain_extract(1, iembT_hbm, iblk_v, irows_v, isem, g)
        return 0

    lax.fori_loop(0, NGRP, grp, 0)

    def tail(g, _):
        vu = ldv(0, g)
        vi = ldv(1, g)
        @pl.when(jnp.any(vu >= EDGE))
        def _():
            for k in range(KF):
                i = g * KF + k
                ru = vu[k]
                @pl.when(ru >= EDGE)
                def _():
                    x = plsc.load_gather(edge_v, [splat(0), iota16,
                                                  splat(ru - EDGE)])
                    plsc.store_scatter(urows_v, [iota16, splat(i)], x)
        @pl.when(jnp.any(vi >= EDGE))
        def _():
            for k in range(KF):
                i = g * KF + k
                ri = vi[k]
                @pl.when(ri >= EDGE)
                def _():
                    x = plsc.load_gather(edge_v, [splat(1), iota16,
                                                  splat(ri - EDGE)])
                    plsc.store_scatter(irows_v, [iota16, splat(i)], x)
        return 0

    lax.fori_loop(0, NGRP, tail, 0)

    pltpu.sync_copy(urows_v, uoutT_hbm.at[:, pl.ds(base, B_PER_W)])
    pltpu.sync_copy(irows_v, ioutT_hbm.at[:, pl.ds(base, B_PER_W)])


@jax.jit
def _sc_gather(uidx, iidx, uembT, iembT):
    mesh = plsc.VectorSubcoreMesh(core_axis_name="c", subcore_axis_name="s")
    out_t = (jax.ShapeDtypeStruct((EMB_D, BATCH), jnp.float32),
             jax.ShapeDtypeStruct((EMB_D, BATCH), jnp.float32))
    fn = functools.partial(
        pl.kernel, mesh=mesh, out_type=out_t,
        compiler_params=pltpu.CompilerParams(needs_layout_passes=False),
        scratch_types=[
            pltpu.VMEM((2, B_PER_W), jnp.int32),
            pltpu.VMEM((KF, EMB_D, 128), jnp.float32),
            pltpu.VMEM((KF, EMB_D, 128), jnp.float32),
            pltpu.VMEM((EMB_D, B_PER_W), jnp.float32),
            pltpu.VMEM((EMB_D, B_PER_W), jnp.float32),
            pltpu.VMEM((2, EMB_D, EDGE_W), jnp.float32),
            pltpu.SemaphoreType.DMA,
            pltpu.SemaphoreType.DMA,
            pltpu.SemaphoreType.DMA,
        ],
    )(_gather_body)
    return fn(uidx, iidx, uembT, iembT)


def _mlp_body(u_ref, i_ref, w1u_ref, w1i_ref, b1_ref, w2_ref, b2_ref,
              w3_ref, b3_ref, o_ref):
    h = (jnp.dot(w1u_ref[...], u_ref[...], preferred_element_type=jnp.float32)
         + jnp.dot(w1i_ref[...], i_ref[...], preferred_element_type=jnp.float32)
         + b1_ref[...])
    h = jnp.maximum(h, 0.0)
    h = jnp.maximum(
        jnp.dot(w2_ref[...], h, preferred_element_type=jnp.float32) + b2_ref[...],
        0.0)
    o_ref[...] = (jnp.dot(w3_ref[...], h, preferred_element_type=jnp.float32)
                  + b3_ref[...])


@jax.jit
def _tc_mlp(uT, iT, w1u, w1i, b1, w2, b2, w3, b3):
    grid = 2
    blk = BATCH // grid
    full = lambda shape: pl.BlockSpec(shape, lambda g: (0,) * len(shape))
    return pl.pallas_call(
        _mlp_body,
        grid=(grid,),
        in_specs=[
            pl.BlockSpec((EMB_D, blk), lambda g: (0, g)),
            pl.BlockSpec((EMB_D, blk), lambda g: (0, g)),
            full((16, EMB_D)), full((16, EMB_D)), full((16, 1)),
            full((8, 16)), full((8, 1)),
            full((1, 8)), full((1, 1)),
        ],
        out_specs=pl.BlockSpec((1, blk), lambda g: (0, g)),
        out_shape=jax.ShapeDtypeStruct((1, BATCH), jnp.float32),
    )(uT, iT, w1u, w1i, b1, w2, b2, w3, b3)


def kernel(user_input, item_input, user_emb, item_emb, W1, b1, W2, b2, W3, b3):
    uidx = user_input.astype(jnp.int32)
    iidx = item_input.astype(jnp.int32)
    uT, iT = _sc_gather(uidx, iidx, user_emb.T, item_emb.T)
    w1u = W1[:, :EMB_D]
    w1i = W1[:, EMB_D:]
    outT = _tc_mlp(uT, iT, w1u, w1i, b1.reshape(16, 1), W2,
                   b2.reshape(8, 1), W3, b3.reshape(1, 1))
    return outT.reshape(BATCH, 1)


# speedup vs baseline: 5.7389x; 1.0065x over previous
import functools

import jax
import jax.numpy as jnp
from jax import lax
from jax.experimental import pallas as pl
from jax.experimental.pallas import tpu as pltpu
from jax.experimental.pallas import tpu_sc as plsc

BATCH = 16384
EMB_D = 16
NROWS = 1000000
NC = 2
NS = 16
NW = NC * NS
B_PER_W = BATCH // NW
KF = 16
NGRP = B_PER_W // KF
LAST_BLK = NROWS // 128 - 1
EDGE = (NROWS // 128) * 128
EDGE_W = NROWS - EDGE


def _gather_body(uidx_hbm, iidx_hbm, uembT_hbm, iembT_hbm,
                 uoutT_hbm, ioutT_hbm,
                 idx_v, ublk_v, iblk_v, urows_v, irows_v, edge_v,
                 usem, isem, esem):
    wid = lax.axis_index("s") * NC + lax.axis_index("c")
    base = wid * B_PER_W
    pltpu.sync_copy(uidx_hbm.at[pl.ds(base, B_PER_W)], idx_v.at[0])
    pltpu.sync_copy(iidx_hbm.at[pl.ds(base, B_PER_W)], idx_v.at[1])
    pltpu.async_copy(uembT_hbm.at[:, pl.ds(EDGE, EDGE_W)], edge_v.at[0],
                     esem).wait()
    pltpu.async_copy(iembT_hbm.at[:, pl.ds(EDGE, EDGE_W)], edge_v.at[1],
                     esem).wait()

    iota16 = lax.iota(jnp.int32, 16)

    def splat(x):
        return jnp.full((16,), x, jnp.int32)

    def ldv(toff, g):
        return idx_v[toff, pl.ds(16 * g, 16)]

    def fire(toff, table, blk_v, sem, g):
        v = ldv(toff, g)
        for k in range(KF):
            r = v[k]
            cb = jnp.minimum(r >> 7, LAST_BLK)
            pltpu.async_copy(table.at[:, pl.ds(cb * 128, 128)],
                             blk_v.at[k], sem)

    def drain_extract(toff, table, blk_v, rows_v, sem, g):
        v = ldv(toff, g)
        for k in range(KF):
            pltpu.make_async_copy(table.at[:, pl.ds(0, 128)],
                                  blk_v.at[k], sem).wait()
        for k in range(KF):
            i = g * KF + k
            x = plsc.load_gather(blk_v, [splat(k), iota16,
                                         splat(v[k] & 127)])
            plsc.store_scatter(rows_v, [iota16, splat(i)], x)

    fire(0, uembT_hbm, ublk_v, usem, 0)

    def grp(g, _):
        fire(1, iembT_hbm, iblk_v, isem, g)
        drain_extract(0, uembT_hbm, ublk_v, urows_v, usem, g)
        @pl.when(g < NGRP - 1)
        def _():
            fire(0, uembT_hbm, ublk_v, usem, g + 1)
        drain_extract(1, iembT_hbm, iblk_v, irows_v, isem, g)
        return 0

    lax.fori_loop(0, NGRP, grp, 0)

    def tail(g, _):
        vu = ldv(0, g)
        vi = ldv(1, g)
        @pl.when(jnp.any(vu >= EDGE))
        def _():
            for k in range(KF):
                i = g * KF + k
                ru = vu[k]
                @pl.when(ru >= EDGE)
                def _():
                    x = plsc.load_gather(edge_v, [splat(0), iota16,
                                                  splat(ru - EDGE)])
                    plsc.store_scatter(urows_v, [iota16, splat(i)], x)
        @pl.when(jnp.any(vi >= EDGE))
        def _():
            for k in range(KF):
                i = g * KF + k
                ri = vi[k]
                @pl.when(ri >= EDGE)
                def _():
                    x = plsc.load_gather(edge_v, [splat(1), iota16,
                                                  splat(ri - EDGE)])
                    plsc.store_scatter(irows_v, [iota16, splat(i)], x)
        return 0

    lax.fori_loop(0, NGRP, tail, 0)

    pltpu.sync_copy(urows_v, uoutT_hbm.at[:, pl.ds(base, B_PER_W)])
    pltpu.sync_copy(irows_v, ioutT_hbm.at[:, pl.ds(base, B_PER_W)])


@jax.jit
def _sc_gather(uidx, iidx, uembT, iembT):
    mesh = plsc.VectorSubcoreMesh(core_axis_name="c", subcore_axis_name="s")
    out_t = (jax.ShapeDtypeStruct((EMB_D, BATCH), jnp.float32),
             jax.ShapeDtypeStruct((EMB_D, BATCH), jnp.float32))
    fn = functools.partial(
        pl.kernel, mesh=mesh, out_type=out_t,
        compiler_params=pltpu.CompilerParams(needs_layout_passes=False),
        scratch_types=[
            pltpu.VMEM((2, B_PER_W), jnp.int32),
            pltpu.VMEM((KF, EMB_D, 128), jnp.float32),
            pltpu.VMEM((KF, EMB_D, 128), jnp.float32),
            pltpu.VMEM((EMB_D, B_PER_W), jnp.float32),
            pltpu.VMEM((EMB_D, B_PER_W), jnp.float32),
            pltpu.VMEM((2, EMB_D, EDGE_W), jnp.float32),
            pltpu.SemaphoreType.DMA,
            pltpu.SemaphoreType.DMA,
            pltpu.SemaphoreType.DMA,
        ],
    )(_gather_body)
    return fn(uidx, iidx, uembT, iembT)


def _mlp_body(u_ref, i_ref, w1u_ref, w1i_ref, b1_ref, w2_ref, b2_ref,
              w3_ref, b3_ref, o_ref):
    h = (jnp.dot(w1u_ref[...], u_ref[...], preferred_element_type=jnp.float32)
         + jnp.dot(w1i_ref[...], i_ref[...], preferred_element_type=jnp.float32)
         + b1_ref[...])
    h = jnp.maximum(h, 0.0)
    h = jnp.maximum(
        jnp.dot(w2_ref[...], h, preferred_element_type=jnp.float32) + b2_ref[...],
        0.0)
    o_ref[...] = (jnp.dot(w3_ref[...], h, preferred_element_type=jnp.float32)
                  + b3_ref[...])


@jax.jit
def _tc_mlp(uT, iT, w1u, w1i, b1, w2, b2, w3, b3):
    grid = 1
    blk = BATCH // grid
    full = lambda shape: pl.BlockSpec(shape, lambda g: (0,) * len(shape))
    return pl.pallas_call(
        _mlp_body,
        grid=(grid,),
        in_specs=[
            pl.BlockSpec((EMB_D, blk), lambda g: (0, g)),
            pl.BlockSpec((EMB_D, blk), lambda g: (0, g)),
            full((16, EMB_D)), full((16, EMB_D)), full((16, 1)),
            full((8, 16)), full((8, 1)),
            full((1, 8)), full((1, 1)),
        ],
        out_specs=pl.BlockSpec((1, blk), lambda g: (0, g)),
        out_shape=jax.ShapeDtypeStruct((1, BATCH), jnp.float32),
    )(uT, iT, w1u, w1i, b1, w2, b2, w3, b3)


def kernel(user_input, item_input, user_emb, item_emb, W1, b1, W2, b2, W3, b3):
    uidx = user_input.astype(jnp.int32)
    iidx = item_input.astype(jnp.int32)
    uT, iT = _sc_gather(uidx, iidx, user_emb.T, item_emb.T)
    w1u = W1[:, :EMB_D]
    w1i = W1[:, EMB_D:]
    outT = _tc_mlp(uT, iT, w1u, w1i, b1.reshape(16, 1), W2,
                   b2.reshape(8, 1), W3, b3.reshape(1, 1))
    return outT.reshape(BATCH, 1)
